# baseline (speedup 1.0000x reference)
import jax
import jax.numpy as jnp
from jax import lax
from jax.experimental import pallas as pl
from jax.experimental.pallas import tpu as pltpu
from jax.experimental.pallas import tpu_sc as plsc

BATCH = 16384
EMBED_DIM = 64
REG_LAMBDA = 0.0001
NW = 32
BPW = BATCH // NW
L = 16
CHUNK = 128
NCHUNK = BPW // CHUNK
ROW = 128


def _sc_body(user_ref, pos_ref, neg_ref, utab_ref, itab_ref, out_ref,
             uidx_v, pidx_v, nidx_v, tux_v, tpx_v, tnx_v,
             ut_v, pt_v, nt_v, out_v, sem):
    wid = lax.axis_index("s") * 2 + lax.axis_index("c")
    base = wid * BPW

    pltpu.sync_copy(user_ref.at[pl.ds(base, BPW)], uidx_v)
    pltpu.sync_copy(pos_ref.at[pl.ds(base, BPW)], pidx_v)
    pltpu.sync_copy(neg_ref.at[pl.ds(base, BPW)], nidx_v)

    for c in range(NCHUNK):
        for g in range(CHUNK // L):
            sl = pl.ds(c * CHUNK + g * L, L)
            dst = pl.ds(g * L, L)
            tux_v[c, dst] = uidx_v[sl]
            tpx_v[c, dst] = pidx_v[sl]
            tnx_v[c, dst] = nidx_v[sl]

    zero = jnp.zeros((L,), jnp.float32)
    lane = lax.iota(jnp.int32, L)
    perms = [lax.iota(jnp.int32, L) ^ (1 << k) for k in range(4)]
    dnums = lax.GatherDimensionNumbers(
        offset_dims=(), collapsed_slice_dims=(0,), start_index_map=(0,))

    def _lane_sum(v):
        for p in perms:
            v = v + lax.gather(v, p[:, None], dnums, (1,),
                               mode=lax.GatherScatterMode.PROMISE_IN_BOUNDS)
        return v

    def group_body(c, g, carry):
        acc_ls, acc_sq = carry
        diffs = zero
        sq = zero
        for j in range(L):
            jj = g * L + j
            us = [ut_v[jj, pl.ds(k * L, L)] for k in range(4)]
            ps = [pt_v[jj, pl.ds(k * L, L)] for k in range(4)]
            nn = [nt_v[jj, pl.ds(k * L, L)] for k in range(4)]
            prod = (us[0] * (ps[0] - nn[0]) + us[1] * (ps[1] - nn[1])
                    + us[2] * (ps[2] - nn[2]) + us[3] * (ps[3] - nn[3]))
            diffs = jnp.where(lane == j, _lane_sum(prod), diffs)
            sq = (sq + us[0] * us[0] + us[1] * us[1] + us[2] * us[2]
                  + us[3] * us[3] + ps[0] * ps[0] + ps[1] * ps[1]
                  + ps[2] * ps[2] + ps[3] * ps[3] + nn[0] * nn[0]
                  + nn[1] * nn[1] + nn[2] * nn[2] + nn[3] * nn[3])
        y = jnp.exp(-jnp.abs(diffs))
        z = y / (y + 2.0)
        z2 = z * z
        poly = 1.0 + z2 * (0.33333333 + z2 * (0.2 + z2 * (0.14285714
                                                          + z2 * 0.11111111)))
        log1py = 2.0 * z * poly
        ls = jnp.minimum(diffs, 0.0) - log1py
        return acc_ls + ls, acc_sq + sq

    def chunk_body(c, carry):
        du = pltpu.async_copy(utab_ref.at[tux_v.at[c]], ut_v, sem)
        dp = pltpu.async_copy(itab_ref.at[tpx_v.at[c]], pt_v, sem)
        dn = pltpu.async_copy(itab_ref.at[tnx_v.at[c]], nt_v, sem)
        du.wait()
        dp.wait()
        dn.wait()
        return lax.fori_loop(0, CHUNK // L,
                             lambda g, cc: group_body(c, g, cc), carry)

    acc_ls, acc_sq = lax.fori_loop(0, NCHUNK, chunk_body, (zero, zero))
    out_v[0, :] = acc_ls
    out_v[1, :] = acc_sq
    pltpu.sync_copy(out_v, out_ref.at[wid])


def _pad_rows(table):
    t3 = table.reshape(125000, 8, EMBED_DIM)
    t3p = jnp.pad(t3, ((0, 0), (0, 0), (0, ROW - EMBED_DIM)))
    return t3p.reshape(1000000, ROW)


def kernel(user, positive, negative, user_table, item_table):
    utabp = _pad_rows(user_table)
    itabp = _pad_rows(item_table)
    mesh = plsc.VectorSubcoreMesh(core_axis_name="c", subcore_axis_name="s")
    partials = pl.kernel(
        _sc_body,
        mesh=mesh,
        out_type=jax.ShapeDtypeStruct((NW, 2, L), jnp.float32),
        scratch_types=[
            pltpu.VMEM((BPW,), jnp.int32),
            pltpu.VMEM((BPW,), jnp.int32),
            pltpu.VMEM((BPW,), jnp.int32),
            pltpu.VMEM((NCHUNK, CHUNK), jnp.int32),
            pltpu.VMEM((NCHUNK, CHUNK), jnp.int32),
            pltpu.VMEM((NCHUNK, CHUNK), jnp.int32),
            pltpu.VMEM((CHUNK, ROW), jnp.float32),
            pltpu.VMEM((CHUNK, ROW), jnp.float32),
            pltpu.VMEM((CHUNK, ROW), jnp.float32),
            pltpu.VMEM((2, L), jnp.float32),
            pltpu.SemaphoreType.DMA,
        ],
    )(user, positive, negative, utabp, itabp)
    bpr_loss = -jnp.sum(partials[:, 0, :]) / BATCH
    reg_loss = REG_LAMBDA * jnp.sum(partials[:, 1, :]) / (2.0 * BATCH)
    return (bpr_loss, reg_loss)


# speedup vs baseline: 1.0335x; 1.0335x over previous
import jax
import jax.numpy as jnp
from jax import lax
from jax.experimental import pallas as pl
from jax.experimental.pallas import tpu as pltpu
from jax.experimental.pallas import tpu_sc as plsc

BATCH = 16384
EMBED_DIM = 64
REG_LAMBDA = 0.0001
NW = 32
BPW = BATCH // NW
L = 16
CHUNK = 128
NCHUNK = BPW // CHUNK
ROW = 128


def _sc_body(user_ref, pos_ref, neg_ref, utab_ref, itab_ref, out_ref,
             uidx_v, pidx_v, nidx_v, tux_v, tpx_v, tnx_v,
             ut_v, pt_v, nt_v, out_v, sem):
    wid = lax.axis_index("s") * 2 + lax.axis_index("c")
    base = wid * BPW

    pltpu.sync_copy(user_ref.at[pl.ds(base, BPW)], uidx_v)
    pltpu.sync_copy(pos_ref.at[pl.ds(base, BPW)], pidx_v)
    pltpu.sync_copy(neg_ref.at[pl.ds(base, BPW)], nidx_v)

    for c in range(NCHUNK):
        for g in range(CHUNK // L):
            sl = pl.ds(c * CHUNK + g * L, L)
            dst = pl.ds(g * L, L)
            tux_v[c, dst] = uidx_v[sl]
            tpx_v[c, dst] = pidx_v[sl]
            tnx_v[c, dst] = nidx_v[sl]

    zero = jnp.zeros((L,), jnp.float32)
    lane = lax.iota(jnp.int32, L)
    perms = [lax.iota(jnp.int32, L) ^ (1 << k) for k in range(4)]
    dnums = lax.GatherDimensionNumbers(
        offset_dims=(), collapsed_slice_dims=(0,), start_index_map=(0,))

    def _lane_sum(v):
        for p in perms:
            v = v + lax.gather(v, p[:, None], dnums, (1,),
                               mode=lax.GatherScatterMode.PROMISE_IN_BOUNDS)
        return v

    def group_body(c, g, carry):
        acc_ls, acc_sq = carry
        diffs = zero
        sq = zero
        for j in range(L):
            jj = g * L + j
            us = [ut_v[jj, pl.ds(k * L, L)] for k in range(4)]
            ps = [pt_v[jj, pl.ds(k * L, L)] for k in range(4)]
            nn = [nt_v[jj, pl.ds(k * L, L)] for k in range(4)]
            prod = (us[0] * (ps[0] - nn[0]) + us[1] * (ps[1] - nn[1])
                    + us[2] * (ps[2] - nn[2]) + us[3] * (ps[3] - nn[3]))
            diffs = jnp.where(lane == j, _lane_sum(prod), diffs)
            sq = (sq + us[0] * us[0] + us[1] * us[1] + us[2] * us[2]
                  + us[3] * us[3] + ps[0] * ps[0] + ps[1] * ps[1]
                  + ps[2] * ps[2] + ps[3] * ps[3] + nn[0] * nn[0]
                  + nn[1] * nn[1] + nn[2] * nn[2] + nn[3] * nn[3])
        y = jnp.exp(-jnp.abs(diffs))
        z = y / (y + 2.0)
        z2 = z * z
        poly = 1.0 + z2 * (0.33333333 + z2 * (0.2 + z2 * (0.14285714
                                                          + z2 * 0.11111111)))
        log1py = 2.0 * z * poly
        ls = jnp.minimum(diffs, 0.0) - log1py
        return acc_ls + ls, acc_sq + sq

    def chunk_body(c, carry):
        du = pltpu.async_copy(utab_ref.at[tux_v.at[c]], ut_v, sem)
        dp = pltpu.async_copy(itab_ref.at[tpx_v.at[c]], pt_v, sem)
        dn = pltpu.async_copy(itab_ref.at[tnx_v.at[c]], nt_v, sem)
        du.wait()
        dp.wait()
        dn.wait()
        return lax.fori_loop(0, CHUNK // L,
                             lambda g, cc: group_body(c, g, cc), carry)

    acc_ls, acc_sq = lax.fori_loop(0, NCHUNK, chunk_body, (zero, zero))
    out_v[0, :] = acc_ls
    out_v[1, :] = acc_sq
    pltpu.sync_copy(out_v, out_ref.at[wid])


def _pad_rows(table):
    return jnp.pad(table, ((0, 0), (0, ROW - EMBED_DIM)))


def kernel(user, positive, negative, user_table, item_table):
    utabp = _pad_rows(user_table)
    itabp = _pad_rows(item_table)
    mesh = plsc.VectorSubcoreMesh(core_axis_name="c", subcore_axis_name="s")
    partials = pl.kernel(
        _sc_body,
        mesh=mesh,
        out_type=jax.ShapeDtypeStruct((NW, 2, L), jnp.float32),
        scratch_types=[
            pltpu.VMEM((BPW,), jnp.int32),
            pltpu.VMEM((BPW,), jnp.int32),
            pltpu.VMEM((BPW,), jnp.int32),
            pltpu.VMEM((NCHUNK, CHUNK), jnp.int32),
            pltpu.VMEM((NCHUNK, CHUNK), jnp.int32),
            pltpu.VMEM((NCHUNK, CHUNK), jnp.int32),
            pltpu.VMEM((CHUNK, ROW), jnp.float32),
            pltpu.VMEM((CHUNK, ROW), jnp.float32),
            pltpu.VMEM((CHUNK, ROW), jnp.float32),
            pltpu.VMEM((2, L), jnp.float32),
            pltpu.SemaphoreType.DMA,
        ],
    )(user, positive, negative, utabp, itabp)
    bpr_loss = -jnp.sum(partials[:, 0, :]) / BATCH
    reg_loss = REG_LAMBDA * jnp.sum(partials[:, 1, :]) / (2.0 * BATCH)
    return (bpr_loss, reg_loss)


# speedup vs baseline: 1.2522x; 1.2116x over previous
import jax
import jax.numpy as jnp
from jax import lax
from jax.experimental import pallas as pl
from jax.experimental.pallas import tpu as pltpu
from jax.experimental.pallas import tpu_sc as plsc

BATCH = 16384
EMBED_DIM = 64
REG_LAMBDA = 0.0001
NW = 32
BPW = BATCH // NW
L = 16
CHUNK = 32
NCHUNK = BPW // CHUNK
ROW = 128
PADBLK = 8000


def _tc_pad_body(in_ref, out_ref):
    out_ref[:, :EMBED_DIM] = in_ref[...]
    out_ref[:, EMBED_DIM:] = jnp.zeros((PADBLK, ROW - EMBED_DIM), jnp.float32)


def _pad_rows_tc(table):
    return pl.pallas_call(
        _tc_pad_body,
        grid=(table.shape[0] // PADBLK,),
        in_specs=[pl.BlockSpec((PADBLK, EMBED_DIM), lambda i: (i, 0))],
        out_specs=pl.BlockSpec((PADBLK, ROW), lambda i: (i, 0)),
        out_shape=jax.ShapeDtypeStruct((table.shape[0], ROW), jnp.float32),
    )(table)


def _sc_body(user_ref, pos_ref, neg_ref, utab_ref, itab_ref, out_ref,
             uidx_v, pidx_v, nidx_v, rux_v, tpx_v, tnx_v,
             ut_v, pt_v, nt_v, out_v, sem):
    wid = lax.axis_index("s") * 2 + lax.axis_index("c")
    base = wid * BPW

    pltpu.sync_copy(user_ref.at[pl.ds(base, BPW)], uidx_v)
    pltpu.sync_copy(pos_ref.at[pl.ds(base, BPW)], pidx_v)
    pltpu.sync_copy(neg_ref.at[pl.ds(base, BPW)], nidx_v)

    for c in range(NCHUNK):
        for g in range(CHUNK // L):
            sl = pl.ds(c * CHUNK + g * L, L)
            dst = pl.ds(g * L, L)
            rux_v[c, dst] = uidx_v[sl]
            tpx_v[c, dst] = pidx_v[sl] >> 3
            tnx_v[c, dst] = nidx_v[sl] >> 3

    zero = jnp.zeros((L,), jnp.float32)
    lane = lax.iota(jnp.int32, L)
    perms = [lax.iota(jnp.int32, L) ^ (1 << k) for k in range(4)]
    dnums = lax.GatherDimensionNumbers(
        offset_dims=(), collapsed_slice_dims=(0,), start_index_map=(0,))

    def _lane_sum(v):
        for p in perms:
            v = v + lax.gather(v, p[:, None], dnums, (1,),
                               mode=lax.GatherScatterMode.PROMISE_IN_BOUNDS)
        return v

    def chunk_body(c, carry):
        acc_ls, acc_sq = carry
        descs = [pltpu.async_copy(utab_ref.at[rux_v.at[c]], ut_v, sem)]
        for g in range(CHUNK // L):
            tpv = tpx_v[c, pl.ds(g * L, L)]
            tnv = tnx_v[c, pl.ds(g * L, L)]
            for j in range(L):
                jj = g * L + j
                descs.append(pltpu.async_copy(itab_ref.at[tpv[j]], pt_v.at[jj], sem))
                descs.append(pltpu.async_copy(itab_ref.at[tnv[j]], nt_v.at[jj], sem))
        for d in descs:
            d.wait()
        for g in range(CHUNK // L):
            pvec = pidx_v[pl.ds(c * CHUNK + g * L, L)]
            nvec = nidx_v[pl.ds(c * CHUNK + g * L, L)]
            diffs = zero
            sq = zero
            for j in range(L):
                jj = g * L + j
                rp = pvec[j] & 7
                rn = nvec[j] & 7
                us = [ut_v[jj, pl.ds(k * L, L)] for k in range(4)]
                ps = [pt_v[jj, rp, pl.ds(k * L, L)] for k in range(4)]
                nn = [nt_v[jj, rn, pl.ds(k * L, L)] for k in range(4)]
                prod = (us[0] * (ps[0] - nn[0]) + us[1] * (ps[1] - nn[1])
                        + us[2] * (ps[2] - nn[2]) + us[3] * (ps[3] - nn[3]))
                diffs = jnp.where(lane == j, _lane_sum(prod), diffs)
                sq = (sq + us[0] * us[0] + us[1] * us[1] + us[2] * us[2]
                      + us[3] * us[3] + ps[0] * ps[0] + ps[1] * ps[1]
                      + ps[2] * ps[2] + ps[3] * ps[3] + nn[0] * nn[0]
                      + nn[1] * nn[1] + nn[2] * nn[2] + nn[3] * nn[3])
            y = jnp.exp(-jnp.abs(diffs))
            z = y / (y + 2.0)
            z2 = z * z
            poly = 1.0 + z2 * (0.33333333 + z2 * (0.2 + z2 * (0.14285714
                                                              + z2 * 0.11111111)))
            log1py = 2.0 * z * poly
            ls = jnp.minimum(diffs, 0.0) - log1py
            acc_ls = acc_ls + ls
            acc_sq = acc_sq + sq
        return acc_ls, acc_sq

    acc_ls, acc_sq = lax.fori_loop(0, NCHUNK, chunk_body, (zero, zero))
    out_v[0, :] = acc_ls
    out_v[1, :] = acc_sq
    pltpu.sync_copy(out_v, out_ref.at[wid])


def kernel(user, positive, negative, user_table, item_table):
    utabp = _pad_rows_tc(user_table)
    itab3 = item_table.reshape(125000, 8, EMBED_DIM)
    mesh = plsc.VectorSubcoreMesh(core_axis_name="c", subcore_axis_name="s")
    partials = pl.kernel(
        _sc_body,
        mesh=mesh,
        out_type=jax.ShapeDtypeStruct((NW, 2, L), jnp.float32),
        scratch_types=[
            pltpu.VMEM((BPW,), jnp.int32),
            pltpu.VMEM((BPW,), jnp.int32),
            pltpu.VMEM((BPW,), jnp.int32),
            pltpu.VMEM((NCHUNK, CHUNK), jnp.int32),
            pltpu.VMEM((NCHUNK, CHUNK), jnp.int32),
            pltpu.VMEM((NCHUNK, CHUNK), jnp.int32),
            pltpu.VMEM((CHUNK, ROW), jnp.float32),
            pltpu.VMEM((CHUNK, 8, EMBED_DIM), jnp.float32),
            pltpu.VMEM((CHUNK, 8, EMBED_DIM), jnp.float32),
            pltpu.VMEM((2, L), jnp.float32),
            pltpu.SemaphoreType.DMA,
        ],
    )(user, positive, negative, utabp, itab3)
    bpr_loss = -jnp.sum(partials[:, 0, :]) / BATCH
    reg_loss = REG_LAMBDA * jnp.sum(partials[:, 1, :]) / (2.0 * BATCH)
    return (bpr_loss, reg_loss)


# speedup vs baseline: 2.0133x; 1.6078x over previous
import jax
import jax.numpy as jnp
from jax import lax
from jax.experimental import pallas as pl
from jax.experimental.pallas import tpu as pltpu
from jax.experimental.pallas import tpu_sc as plsc

BATCH = 16384
EMBED_DIM = 64
REG_LAMBDA = 0.0001
NW = 32
BPW = BATCH // NW
L = 16
CHUNK = 16
NCHUNK = BPW // CHUNK


def _sc_body(user_ref, pos_ref, neg_ref, utab_ref, itab_ref, out_ref,
             uidx_v, pidx_v, nidx_v, tux_v, tpx_v, tnx_v,
             ut_a, pt_a, nt_a, ut_b, pt_b, nt_b, out_v, sem_a, sem_b):
    wid = lax.axis_index("s") * 2 + lax.axis_index("c")
    base = wid * BPW

    pltpu.sync_copy(user_ref.at[pl.ds(base, BPW)], uidx_v)
    pltpu.sync_copy(pos_ref.at[pl.ds(base, BPW)], pidx_v)
    pltpu.sync_copy(neg_ref.at[pl.ds(base, BPW)], nidx_v)

    for c in range(NCHUNK):
        sl = pl.ds(c * CHUNK, L)
        tux_v[c, :] = uidx_v[sl] >> 3
        tpx_v[c, :] = pidx_v[sl] >> 3
        tnx_v[c, :] = nidx_v[sl] >> 3

    zero = jnp.zeros((L,), jnp.float32)
    lane = lax.iota(jnp.int32, L)
    perms = [lax.iota(jnp.int32, L) ^ (1 << k) for k in range(4)]
    dnums = lax.GatherDimensionNumbers(
        offset_dims=(), collapsed_slice_dims=(0,), start_index_map=(0,))

    def _lane_sum(v):
        for p in perms:
            v = v + lax.gather(v, p[:, None], dnums, (1,),
                               mode=lax.GatherScatterMode.PROMISE_IN_BOUNDS)
        return v

    def fire(c, ut_v, pt_v, nt_v, sem):
        tuv = tux_v[c, :]
        tpv = tpx_v[c, :]
        tnv = tnx_v[c, :]
        for j in range(L):
            pltpu.async_copy(utab_ref.at[tuv[j]], ut_v.at[j], sem)
            pltpu.async_copy(itab_ref.at[tpv[j]], pt_v.at[j], sem)
            pltpu.async_copy(itab_ref.at[tnv[j]], nt_v.at[j], sem)

    def drain(ut_v, pt_v, nt_v, sem):
        pltpu.make_async_copy(utab_ref.at[pl.ds(0, CHUNK)], ut_v, sem).wait()
        pltpu.make_async_copy(utab_ref.at[pl.ds(0, CHUNK)], pt_v, sem).wait()
        pltpu.make_async_copy(utab_ref.at[pl.ds(0, CHUNK)], nt_v, sem).wait()

    def compute(c, ut_v, pt_v, nt_v, carry):
        acc_ls, acc_sq = carry
        uvec = uidx_v[pl.ds(c * CHUNK, L)]
        pvec = pidx_v[pl.ds(c * CHUNK, L)]
        nvec = nidx_v[pl.ds(c * CHUNK, L)]
        diffs = zero
        sq = zero
        for j in range(L):
            ru = uvec[j] & 7
            rp = pvec[j] & 7
            rn = nvec[j] & 7
            us = [ut_v[j, ru, pl.ds(k * L, L)] for k in range(4)]
            ps = [pt_v[j, rp, pl.ds(k * L, L)] for k in range(4)]
            nn = [nt_v[j, rn, pl.ds(k * L, L)] for k in range(4)]
            prod = (us[0] * (ps[0] - nn[0]) + us[1] * (ps[1] - nn[1])
                    + us[2] * (ps[2] - nn[2]) + us[3] * (ps[3] - nn[3]))
            diffs = jnp.where(lane == j, _lane_sum(prod), diffs)
            sq = (sq + us[0] * us[0] + us[1] * us[1] + us[2] * us[2]
                  + us[3] * us[3] + ps[0] * ps[0] + ps[1] * ps[1]
                  + ps[2] * ps[2] + ps[3] * ps[3] + nn[0] * nn[0]
                  + nn[1] * nn[1] + nn[2] * nn[2] + nn[3] * nn[3])
        y = jnp.exp(-jnp.abs(diffs))
        z = y / (y + 2.0)
        z2 = z * z
        poly = 1.0 + z2 * (0.33333333 + z2 * (0.2 + z2 * (0.14285714
                                                          + z2 * 0.11111111)))
        log1py = 2.0 * z * poly
        ls = jnp.minimum(diffs, 0.0) - log1py
        return acc_ls + ls, acc_sq + sq

    fire(0, ut_a, pt_a, nt_a, sem_a)

    def body2(i, carry):
        c0 = 2 * i
        drain(ut_a, pt_a, nt_a, sem_a)
        fire(c0 + 1, ut_b, pt_b, nt_b, sem_b)
        carry = compute(c0, ut_a, pt_a, nt_a, carry)
        drain(ut_b, pt_b, nt_b, sem_b)

        @pl.when(c0 + 2 < NCHUNK)
        def _():
            fire(c0 + 2, ut_a, pt_a, nt_a, sem_a)

        carry = compute(c0 + 1, ut_b, pt_b, nt_b, carry)
        return carry

    acc_ls, acc_sq = lax.fori_loop(0, NCHUNK // 2, body2, (zero, zero))
    out_v[0, :] = acc_ls
    out_v[1, :] = acc_sq
    pltpu.sync_copy(out_v, out_ref.at[wid])


def kernel(user, positive, negative, user_table, item_table):
    utab3 = user_table.reshape(125000, 8, EMBED_DIM)
    itab3 = item_table.reshape(125000, 8, EMBED_DIM)
    mesh = plsc.VectorSubcoreMesh(core_axis_name="c", subcore_axis_name="s")
    tile_t = pltpu.VMEM((CHUNK, 8, EMBED_DIM), jnp.float32)
    partials = pl.kernel(
        _sc_body,
        mesh=mesh,
        out_type=jax.ShapeDtypeStruct((NW, 2, L), jnp.float32),
        scratch_types=[
            pltpu.VMEM((BPW,), jnp.int32),
            pltpu.VMEM((BPW,), jnp.int32),
            pltpu.VMEM((BPW,), jnp.int32),
            pltpu.VMEM((NCHUNK, CHUNK), jnp.int32),
            pltpu.VMEM((NCHUNK, CHUNK), jnp.int32),
            pltpu.VMEM((NCHUNK, CHUNK), jnp.int32),
            tile_t, tile_t, tile_t, tile_t, tile_t, tile_t,
            pltpu.VMEM((2, L), jnp.float32),
            pltpu.SemaphoreType.DMA,
            pltpu.SemaphoreType.DMA,
        ],
    )(user, positive, negative, utab3, itab3)
    bpr_loss = -jnp.sum(partials[:, 0, :]) / BATCH
    reg_loss = REG_LAMBDA * jnp.sum(partials[:, 1, :]) / (2.0 * BATCH)
    return (bpr_loss, reg_loss)
